# baseline (speedup 1.0000x reference)
import functools

import jax
import jax.numpy as jnp
from jax import lax
from jax.experimental import pallas as pl
from jax.experimental.pallas import tpu as pltpu
from jax.experimental.pallas import tpu_sc as plsc

NC = 2
NS = 16
NW = NC * NS
CH = 128
DEGW = 16

K0 = 80
K1 = 80
KH0 = K0 // 2
KH1 = K1 // 2
KHM = max(KH0, KH1)
KR = 2 * KHM


def _mesh():
    return plsc.VectorSubcoreMesh(core_axis_name="c", subcore_axis_name="s")


def _deg_kernel(n_pad):
    RPT = n_pad // NS

    @functools.partial(
        pl.kernel,
        mesh=_mesh(),
        compiler_params=pltpu.CompilerParams(use_tc_tiling_on_sc=False),
        out_type=jax.ShapeDtypeStruct((NC, n_pad, DEGW), jnp.float32),
        scratch_types=[
            pltpu.VMEM((KR, CH), jnp.int32),
            pltpu.VMEM((CH, DEGW), jnp.float32),
            pltpu.VMEM_SHARED((n_pad, DEGW), jnp.float32),
        ],
    )
    def deg_k(didx_hbm, ones_hbm, zeros_hbm, out_hbm, didx_v, ones_v, deg_sh):
        c = lax.axis_index("c")
        s = lax.axis_index("s")
        wid = s * NC + c
        kc = jnp.where(c == 0, K0, K1)
        pltpu.sync_copy(ones_hbm, ones_v)
        pltpu.sync_copy(zeros_hbm, deg_sh.at[pl.ds(s * RPT, RPT)])
        pltpu.sync_copy(didx_hbm.at[wid], didx_v)
        plsc.subcore_barrier()

        def step(k, carry):
            pltpu.sync_copy(ones_v, deg_sh.at[didx_v.at[k]], add=True)
            return carry

        lax.fori_loop(0, kc, step, 0)
        plsc.subcore_barrier()
        pltpu.sync_copy(deg_sh.at[pl.ds(s * RPT, RPT)],
                        out_hbm.at[c, pl.ds(s * RPT, RPT)])

    return deg_k


def _agg_kernel(n_pad, D):
    RPT = n_pad // NS

    @functools.partial(
        pl.kernel,
        mesh=_mesh(),
        compiler_params=pltpu.CompilerParams(use_tc_tiling_on_sc=False),
        out_type=jax.ShapeDtypeStruct((NC, n_pad, D), jnp.float32),
        scratch_types=[
            pltpu.VMEM((KHM, CH), jnp.int32),
            pltpu.VMEM((KHM, CH), jnp.int32),
            pltpu.VMEM((CH, D), jnp.float32),
            pltpu.VMEM_SHARED((n_pad, D), jnp.float32),
            pltpu.SemaphoreType.DMA,
        ],
    )
    def agg_k(tab_hbm, sidx_hbm, didx_hbm, zeros_hbm, out_hbm,
              sidx_v, didx_v, rows_v, acc_sh, gsem):
        c = lax.axis_index("c")
        s = lax.axis_index("s")
        wid = s * NC + c
        khc = jnp.where(c == 0, KH0, KH1)
        pltpu.sync_copy(zeros_hbm, acc_sh.at[pl.ds(s * RPT, RPT)])
        plsc.subcore_barrier()

        for ph in range(2):
            pltpu.sync_copy(sidx_hbm.at[wid, pl.ds(ph * khc, KHM)], sidx_v)
            pltpu.sync_copy(didx_hbm.at[wid, pl.ds(ph * khc, KHM)], didx_v)

            def step(k, carry):
                pltpu.async_copy(tab_hbm.at[sidx_v.at[k]], rows_v,
                                 gsem).wait()
                pltpu.sync_copy(rows_v, acc_sh.at[didx_v.at[k]], add=True)
                return carry

            lax.fori_loop(0, khc, step, 0)

        plsc.subcore_barrier()
        pltpu.sync_copy(acc_sh.at[pl.ds(s * RPT, RPT)],
                        out_hbm.at[c, pl.ds(s * RPT, RPT)])

    return agg_k


def _stage1_body(x_ref, w_ref, degA_ref, degB_ref, hs_ref, dinv_ref):
    deg = degA_ref[...] + degB_ref[...] + 1.0
    dinv = lax.rsqrt(deg)
    h = jnp.dot(x_ref[...], w_ref[...], preferred_element_type=jnp.float32)
    hs_ref[...] = h * dinv
    dinv_ref[...] = dinv


def _stage2_body(accA_ref, accB_ref, hs1_ref, dinv_ref, w_ref, b_ref, hs2_ref):
    dinv = dinv_ref[...]
    z = (accA_ref[...] + accB_ref[...]) * dinv + hs1_ref[...] * dinv + b_ref[...]
    z = jnp.maximum(z, 0.0)
    hs2_ref[...] = jnp.dot(z, w_ref[...], preferred_element_type=jnp.float32) * dinv


def _stage3_body(n_rows, n_grid,
                 accA_ref, accB_ref, hs2_ref, dinv_ref, b_ref, wfcT_ref,
                 bfc_ref, out_ref, acc_scr):
    i = pl.program_id(0)

    @pl.when(i == 0)
    def _():
        acc_scr[...] = jnp.zeros_like(acc_scr)

    dinv = dinv_ref[...]
    z = (accA_ref[...] + accB_ref[...]) * dinv + hs2_ref[...] * dinv + b_ref[...]
    z = jnp.maximum(z, 0.0)
    acc_scr[...] += jnp.sum(z, axis=0, keepdims=True)

    @pl.when(i == n_grid - 1)
    def _():
        g = acc_scr[...] * (1.0 / n_rows)
        out_ref[...] = (jnp.sum(g * wfcT_ref[...], axis=1, keepdims=True)
                        + bfc_ref[...])


def kernel(x, edge_index, W1, b1, W2, b2, Wfc, bfc):
    N, D_IN = x.shape
    DH = W1.shape[1]
    DO = W2.shape[1]
    E = edge_index.shape[1]

    E_cap = NS * (K0 + K1) * CH
    pad = E_cap - E
    src = jnp.concatenate([edge_index[0], jnp.zeros((pad,), jnp.int32)])
    dst = jnp.concatenate([edge_index[1], jnp.full((pad,), N, jnp.int32)])
    n0 = NS * K0 * CH

    def _layout(v, fill, off1):
        a = jnp.full((NS, NC, KR, CH), fill, jnp.int32)
        a = a.at[:, 0, :K0].set(v[:n0].reshape(NS, K0, CH))
        a = a.at[:, 1, :K1].set(v[n0:].reshape(NS, K1, CH) + off1)
        return a.reshape(NW, KR, CH)

    s3 = _layout(src, 0, N)
    d3 = _layout(dst, N, 0)

    RPT = -(-(N + 1) // (NS * 8)) * 8
    n_pad = RPT * NS

    ones_deg = jnp.ones((CH, DEGW), jnp.float32)
    z_deg = jnp.zeros((RPT, DEGW), jnp.float32)
    z_h = jnp.zeros((RPT, DH), jnp.float32)
    z_o = jnp.zeros((RPT, DO), jnp.float32)

    degout = _deg_kernel(n_pad)(d3, ones_deg, z_deg)
    degA = degout[0, :N, 0:1]
    degB = degout[1, :N, 0:1]

    R = 2000
    NG = N // R
    hs1, dinv = pl.pallas_call(
        _stage1_body,
        grid=(NG,),
        in_specs=[
            pl.BlockSpec((R, D_IN), lambda i: (i, 0)),
            pl.BlockSpec((D_IN, DH), lambda i: (0, 0)),
            pl.BlockSpec((R, 1), lambda i: (i, 0)),
            pl.BlockSpec((R, 1), lambda i: (i, 0)),
        ],
        out_specs=[
            pl.BlockSpec((R, DH), lambda i: (i, 0)),
            pl.BlockSpec((R, 1), lambda i: (i, 0)),
        ],
        out_shape=[
            jax.ShapeDtypeStruct((N, DH), jnp.float32),
            jax.ShapeDtypeStruct((N, 1), jnp.float32),
        ],
    )(x, W1, degA, degB)

    acc1 = _agg_kernel(n_pad, DH)(jnp.concatenate([hs1, hs1]), s3, d3, z_h)

    hs2 = pl.pallas_call(
        _stage2_body,
        grid=(NG,),
        in_specs=[
            pl.BlockSpec((R, DH), lambda i: (i, 0)),
            pl.BlockSpec((R, DH), lambda i: (i, 0)),
            pl.BlockSpec((R, DH), lambda i: (i, 0)),
            pl.BlockSpec((R, 1), lambda i: (i, 0)),
            pl.BlockSpec((DH, DO), lambda i: (0, 0)),
            pl.BlockSpec((1, DH), lambda i: (0, 0)),
        ],
        out_specs=pl.BlockSpec((R, DO), lambda i: (i, 0)),
        out_shape=jax.ShapeDtypeStruct((N, DO), jnp.float32),
    )(acc1[0, :N], acc1[1, :N], hs1, dinv, W2, b1.reshape(1, DH))

    acc2 = _agg_kernel(n_pad, DO)(jnp.concatenate([hs2, hs2]), s3, d3, z_o)

    out = pl.pallas_call(
        functools.partial(_stage3_body, float(N), NG),
        grid=(NG,),
        in_specs=[
            pl.BlockSpec((R, DO), lambda i: (i, 0)),
            pl.BlockSpec((R, DO), lambda i: (i, 0)),
            pl.BlockSpec((R, DO), lambda i: (i, 0)),
            pl.BlockSpec((R, 1), lambda i: (i, 0)),
            pl.BlockSpec((1, DO), lambda i: (0, 0)),
            pl.BlockSpec((1, DO), lambda i: (0, 0)),
            pl.BlockSpec((1, 1), lambda i: (0, 0)),
        ],
        out_specs=pl.BlockSpec((1, 1), lambda i: (0, 0)),
        out_shape=jax.ShapeDtypeStruct((1, 1), jnp.float32),
        scratch_shapes=[pltpu.VMEM((1, DO), jnp.float32)],
    )(acc2[0, :N], acc2[1, :N], hs2, dinv, b2.reshape(1, DO),
      Wfc.reshape(1, DO), bfc.reshape(1, 1))

    return out


# speedup vs baseline: 1.7897x; 1.7897x over previous
import functools

import jax
import jax.numpy as jnp
from jax import lax
from jax.experimental import pallas as pl
from jax.experimental.pallas import tpu as pltpu
from jax.experimental.pallas import tpu_sc as plsc

NC = 2
NS = 16
NW = NC * NS
CH = 128
DEGW = 16

K0 = 110
K1 = 48
KH0 = K0 // 2
KH1 = K1 // 2
KHM = max(KH0, KH1)
KR = 2 * KHM


def _mesh():
    return plsc.VectorSubcoreMesh(core_axis_name="c", subcore_axis_name="s")


def _deg_kernel(n_pad):
    RPT = n_pad // NS

    @functools.partial(
        pl.kernel,
        mesh=_mesh(),
        compiler_params=pltpu.CompilerParams(use_tc_tiling_on_sc=False),
        out_type=jax.ShapeDtypeStruct((NC, n_pad, DEGW), jnp.float32),
        scratch_types=[
            pltpu.VMEM((KR, CH), jnp.int32),
            pltpu.VMEM((CH, DEGW), jnp.float32),
            pltpu.VMEM_SHARED((n_pad, DEGW), jnp.float32),
        ],
    )
    def deg_k(didx_hbm, ones_hbm, zeros_hbm, out_hbm, didx_v, ones_v, deg_sh):
        c = lax.axis_index("c")
        s = lax.axis_index("s")
        wid = s * NC + c
        kc = jnp.where(c == 0, K0, K1)
        pltpu.sync_copy(ones_hbm, ones_v)
        pltpu.sync_copy(zeros_hbm, deg_sh.at[pl.ds(s * RPT, RPT)])
        pltpu.sync_copy(didx_hbm.at[wid], didx_v)
        plsc.subcore_barrier()

        def step(k, carry):
            pltpu.sync_copy(ones_v, deg_sh.at[didx_v.at[k]], add=True)
            return carry

        lax.fori_loop(0, kc, step, 0)
        plsc.subcore_barrier()
        pltpu.sync_copy(deg_sh.at[pl.ds(s * RPT, RPT)],
                        out_hbm.at[c, pl.ds(s * RPT, RPT)])

    return deg_k


def _agg_kernel(n_pad, D):
    RPT = n_pad // NS

    @functools.partial(
        pl.kernel,
        mesh=_mesh(),
        compiler_params=pltpu.CompilerParams(use_tc_tiling_on_sc=False),
        out_type=jax.ShapeDtypeStruct((NC, n_pad, D), jnp.float32),
        scratch_types=[
            pltpu.VMEM((KHM, CH), jnp.int32),
            pltpu.VMEM((KHM, CH), jnp.int32),
            pltpu.VMEM((CH, D), jnp.float32),
            pltpu.VMEM_SHARED((n_pad, D), jnp.float32),
            pltpu.SemaphoreType.DMA,
        ],
    )
    def agg_k(tab_hbm, sidx_hbm, didx_hbm, zeros_hbm, out_hbm,
              sidx_v, didx_v, rows_v, acc_sh, gsem):
        c = lax.axis_index("c")
        s = lax.axis_index("s")
        wid = s * NC + c
        khc = jnp.where(c == 0, KH0, KH1)
        pltpu.sync_copy(zeros_hbm, acc_sh.at[pl.ds(s * RPT, RPT)])
        plsc.subcore_barrier()

        for ph in range(2):
            pltpu.sync_copy(sidx_hbm.at[wid, pl.ds(ph * khc, KHM)], sidx_v)
            pltpu.sync_copy(didx_hbm.at[wid, pl.ds(ph * khc, KHM)], didx_v)

            def step(k, carry):
                pltpu.async_copy(tab_hbm.at[sidx_v.at[k]], rows_v,
                                 gsem).wait()
                pltpu.sync_copy(rows_v, acc_sh.at[didx_v.at[k]], add=True)
                return carry

            lax.fori_loop(0, khc, step, 0)

        plsc.subcore_barrier()
        pltpu.sync_copy(acc_sh.at[pl.ds(s * RPT, RPT)],
                        out_hbm.at[c, pl.ds(s * RPT, RPT)])

    return agg_k


def _stage1_body(x_ref, w_ref, degA_ref, degB_ref, hs_ref, dinv_ref):
    deg = degA_ref[...] + degB_ref[...] + 1.0
    dinv = lax.rsqrt(deg)
    h = jnp.dot(x_ref[...], w_ref[...], preferred_element_type=jnp.float32)
    hs_ref[...] = h * dinv
    dinv_ref[...] = dinv


def _stage2_body(accA_ref, accB_ref, hs1_ref, dinv_ref, w_ref, b_ref, hs2_ref):
    dinv = dinv_ref[...]
    z = (accA_ref[...] + accB_ref[...]) * dinv + hs1_ref[...] * dinv + b_ref[...]
    z = jnp.maximum(z, 0.0)
    hs2_ref[...] = jnp.dot(z, w_ref[...], preferred_element_type=jnp.float32) * dinv


def _stage3_body(n_rows, n_grid,
                 accA_ref, accB_ref, hs2_ref, dinv_ref, b_ref, wfcT_ref,
                 bfc_ref, out_ref, acc_scr):
    i = pl.program_id(0)

    @pl.when(i == 0)
    def _():
        acc_scr[...] = jnp.zeros_like(acc_scr)

    dinv = dinv_ref[...]
    z = (accA_ref[...] + accB_ref[...]) * dinv + hs2_ref[...] * dinv + b_ref[...]
    z = jnp.maximum(z, 0.0)
    acc_scr[...] += jnp.sum(z, axis=0, keepdims=True)

    @pl.when(i == n_grid - 1)
    def _():
        g = acc_scr[...] * (1.0 / n_rows)
        out_ref[...] = (jnp.sum(g * wfcT_ref[...], axis=1, keepdims=True)
                        + bfc_ref[...])


def kernel(x, edge_index, W1, b1, W2, b2, Wfc, bfc):
    N, D_IN = x.shape
    DH = W1.shape[1]
    DO = W2.shape[1]
    E = edge_index.shape[1]

    E_cap = NS * (K0 + K1) * CH
    pad = E_cap - E
    src = jnp.concatenate([edge_index[0], jnp.zeros((pad,), jnp.int32)])
    dst = jnp.concatenate([edge_index[1], jnp.full((pad,), N, jnp.int32)])
    n0 = NS * K0 * CH

    def _layout(v, fill):
        a = jnp.full((NS, NC, KR, CH), fill, jnp.int32)
        a = a.at[:, 0, :K0].set(v[:n0].reshape(NS, K0, CH))
        a = a.at[:, 1, :K1].set(v[n0:].reshape(NS, K1, CH))
        return a.reshape(NW, KR, CH)

    s3 = _layout(src, 0)
    d3 = _layout(dst, N)

    RPT = -(-(N + 1) // (NS * 8)) * 8
    n_pad = RPT * NS

    ones_deg = jnp.ones((CH, DEGW), jnp.float32)
    z_deg = jnp.zeros((RPT, DEGW), jnp.float32)
    z_h = jnp.zeros((RPT, DH), jnp.float32)
    z_o = jnp.zeros((RPT, DO), jnp.float32)

    degout = _deg_kernel(n_pad)(d3, ones_deg, z_deg)
    degA = degout[0, :N, 0:1]
    degB = degout[1, :N, 0:1]

    R = 2000
    NG = N // R
    hs1, dinv = pl.pallas_call(
        _stage1_body,
        grid=(NG,),
        in_specs=[
            pl.BlockSpec((R, D_IN), lambda i: (i, 0)),
            pl.BlockSpec((D_IN, DH), lambda i: (0, 0)),
            pl.BlockSpec((R, 1), lambda i: (i, 0)),
            pl.BlockSpec((R, 1), lambda i: (i, 0)),
        ],
        out_specs=[
            pl.BlockSpec((R, DH), lambda i: (i, 0)),
            pl.BlockSpec((R, 1), lambda i: (i, 0)),
        ],
        out_shape=[
            jax.ShapeDtypeStruct((N, DH), jnp.float32),
            jax.ShapeDtypeStruct((N, 1), jnp.float32),
        ],
    )(x, W1, degA, degB)

    acc1 = _agg_kernel(n_pad, DH)(hs1, s3, d3, z_h)

    hs2 = pl.pallas_call(
        _stage2_body,
        grid=(NG,),
        in_specs=[
            pl.BlockSpec((R, DH), lambda i: (i, 0)),
            pl.BlockSpec((R, DH), lambda i: (i, 0)),
            pl.BlockSpec((R, DH), lambda i: (i, 0)),
            pl.BlockSpec((R, 1), lambda i: (i, 0)),
            pl.BlockSpec((DH, DO), lambda i: (0, 0)),
            pl.BlockSpec((1, DH), lambda i: (0, 0)),
        ],
        out_specs=pl.BlockSpec((R, DO), lambda i: (i, 0)),
        out_shape=jax.ShapeDtypeStruct((N, DO), jnp.float32),
    )(acc1[0, :N], acc1[1, :N], hs1, dinv, W2, b1.reshape(1, DH))

    acc2 = _agg_kernel(n_pad, DO)(hs2, s3, d3, z_o)

    out = pl.pallas_call(
        functools.partial(_stage3_body, float(N), NG),
        grid=(NG,),
        in_specs=[
            pl.BlockSpec((R, DO), lambda i: (i, 0)),
            pl.BlockSpec((R, DO), lambda i: (i, 0)),
            pl.BlockSpec((R, DO), lambda i: (i, 0)),
            pl.BlockSpec((R, 1), lambda i: (i, 0)),
            pl.BlockSpec((1, DO), lambda i: (0, 0)),
            pl.BlockSpec((1, DO), lambda i: (0, 0)),
            pl.BlockSpec((1, 1), lambda i: (0, 0)),
        ],
        out_specs=pl.BlockSpec((1, 1), lambda i: (0, 0)),
        out_shape=jax.ShapeDtypeStruct((1, 1), jnp.float32),
        scratch_shapes=[pltpu.VMEM((1, DO), jnp.float32)],
    )(acc2[0, :N], acc2[1, :N], hs2, dinv, b2.reshape(1, DO),
      Wfc.reshape(1, DO), bfc.reshape(1, 1))

    return out


# speedup vs baseline: 1.8368x; 1.0263x over previous
import functools

import jax
import jax.numpy as jnp
from jax import lax
from jax.experimental import pallas as pl
from jax.experimental.pallas import tpu as pltpu
from jax.experimental.pallas import tpu_sc as plsc

NC = 2
NS = 16
NW = NC * NS
CH = 128
DEGW = 16

K0 = 110
K1 = 48
KH0 = K0 // 2
KH1 = K1 // 2
KHM = max(KH0, KH1)
KR = 2 * KHM


def _mesh():
    return plsc.VectorSubcoreMesh(core_axis_name="c", subcore_axis_name="s")


def _deg_kernel(n_pad):
    RPT = n_pad // NS

    @functools.partial(
        pl.kernel,
        mesh=_mesh(),
        compiler_params=pltpu.CompilerParams(use_tc_tiling_on_sc=False),
        out_type=jax.ShapeDtypeStruct((NC, n_pad, DEGW), jnp.float32),
        scratch_types=[
            pltpu.VMEM((KR, CH), jnp.int32),
            pltpu.VMEM((CH, DEGW), jnp.float32),
            pltpu.VMEM_SHARED((n_pad, DEGW), jnp.float32),
        ],
    )
    def deg_k(didx_hbm, ones_hbm, zeros_hbm, out_hbm, didx_v, ones_v, deg_sh):
        c = lax.axis_index("c")
        s = lax.axis_index("s")
        wid = s * NC + c
        kc = jnp.where(c == 0, K0, K1)
        pltpu.sync_copy(ones_hbm, ones_v)
        pltpu.sync_copy(zeros_hbm, deg_sh.at[pl.ds(s * RPT, RPT)])
        pltpu.sync_copy(didx_hbm.at[wid], didx_v)
        plsc.subcore_barrier()

        def step(k, carry):
            pltpu.sync_copy(ones_v, deg_sh.at[didx_v.at[k]], add=True)
            return carry

        lax.fori_loop(0, kc, step, 0)
        plsc.subcore_barrier()
        pltpu.sync_copy(deg_sh.at[pl.ds(s * RPT, RPT)],
                        out_hbm.at[c, pl.ds(s * RPT, RPT)])

    return deg_k


def _agg_kernel(n_pad, D):
    RPT = n_pad // NS

    @functools.partial(
        pl.kernel,
        mesh=_mesh(),
        compiler_params=pltpu.CompilerParams(use_tc_tiling_on_sc=False),
        out_type=jax.ShapeDtypeStruct((NC, n_pad, D), jnp.float32),
        scratch_types=[
            pltpu.VMEM((KHM, CH), jnp.int32),
            pltpu.VMEM((KHM, CH), jnp.int32),
            pltpu.VMEM((CH, D), jnp.float32),
            pltpu.VMEM_SHARED((n_pad, D), jnp.float32),
            pltpu.SemaphoreType.DMA,
        ],
    )
    def agg_k(tab_hbm, sidx_hbm, didx_hbm, zeros_hbm, out_hbm,
              sidx_v, didx_v, rows_v, acc_sh, gsem):
        c = lax.axis_index("c")
        s = lax.axis_index("s")
        wid = s * NC + c
        khc = jnp.where(c == 0, KH0, KH1)
        pltpu.sync_copy(zeros_hbm, acc_sh.at[pl.ds(s * RPT, RPT)])
        plsc.subcore_barrier()

        for ph in range(2):
            pltpu.sync_copy(sidx_hbm.at[wid, pl.ds(ph * khc, KHM)], sidx_v)
            pltpu.sync_copy(didx_hbm.at[wid, pl.ds(ph * khc, KHM)], didx_v)

            def step(k, carry):
                pltpu.async_copy(tab_hbm.at[sidx_v.at[k]], rows_v,
                                 gsem).wait()
                pltpu.sync_copy(rows_v, acc_sh.at[didx_v.at[k]], add=True)
                return carry

            lax.fori_loop(0, khc, step, 0)

        plsc.subcore_barrier()
        pltpu.sync_copy(acc_sh.at[pl.ds(s * RPT, RPT)],
                        out_hbm.at[c, pl.ds(s * RPT, RPT)])

    return agg_k


def _stage1_body(x_ref, w_ref, deg_ref, hs_ref, dinv_ref):
    deg = deg_ref[0, :, 0:1] + deg_ref[1, :, 0:1] + 1.0
    dinv = lax.rsqrt(deg)
    h = jnp.dot(x_ref[...], w_ref[...], preferred_element_type=jnp.float32)
    hs_ref[...] = h * dinv
    dinv_ref[...] = dinv


def _stage2_body(acc_ref, hs1_ref, dinv_ref, w_ref, b_ref, hs2_ref):
    dinv = dinv_ref[...]
    z = (acc_ref[0] + acc_ref[1]) * dinv + hs1_ref[...] * dinv + b_ref[...]
    z = jnp.maximum(z, 0.0)
    hs2_ref[...] = jnp.dot(z, w_ref[...], preferred_element_type=jnp.float32) * dinv


def _stage3_body(n_rows, n_grid,
                 acc_ref, hs2_ref, dinv_ref, b_ref, wfcT_ref,
                 bfc_ref, out_ref, acc_scr):
    i = pl.program_id(0)

    @pl.when(i == 0)
    def _():
        acc_scr[...] = jnp.zeros_like(acc_scr)

    dinv = dinv_ref[...]
    z = (acc_ref[0] + acc_ref[1]) * dinv + hs2_ref[...] * dinv + b_ref[...]
    z = jnp.maximum(z, 0.0)
    acc_scr[...] += jnp.sum(z, axis=0, keepdims=True)

    @pl.when(i == n_grid - 1)
    def _():
        g = acc_scr[...] * (1.0 / n_rows)
        out_ref[...] = (jnp.sum(g * wfcT_ref[...], axis=1, keepdims=True)
                        + bfc_ref[...])


def kernel(x, edge_index, W1, b1, W2, b2, Wfc, bfc):
    N, D_IN = x.shape
    DH = W1.shape[1]
    DO = W2.shape[1]
    E = edge_index.shape[1]

    E_cap = NS * (K0 + K1) * CH
    pad = E_cap - E
    src = jnp.concatenate([edge_index[0], jnp.zeros((pad,), jnp.int32)])
    dst = jnp.concatenate([edge_index[1], jnp.full((pad,), N, jnp.int32)])
    n0 = NS * K0 * CH

    def _layout(v, fill):
        a = jnp.full((NS, NC, KR, CH), fill, jnp.int32)
        a = a.at[:, 0, :K0].set(v[:n0].reshape(NS, K0, CH))
        a = a.at[:, 1, :K1].set(v[n0:].reshape(NS, K1, CH))
        return a.reshape(NW, KR, CH)

    s3 = _layout(src, 0)
    d3 = _layout(dst, N)

    RPT = -(-(N + 1) // (NS * 8)) * 8
    n_pad = RPT * NS

    ones_deg = jnp.ones((CH, DEGW), jnp.float32)
    z_deg = jnp.zeros((RPT, DEGW), jnp.float32)
    z_h = jnp.zeros((RPT, DH), jnp.float32)
    z_o = jnp.zeros((RPT, DO), jnp.float32)

    degout = _deg_kernel(n_pad)(d3, ones_deg, z_deg)

    R = 2000
    NG = N // R
    hs1, dinv = pl.pallas_call(
        _stage1_body,
        grid=(NG,),
        in_specs=[
            pl.BlockSpec((R, D_IN), lambda i: (i, 0)),
            pl.BlockSpec((D_IN, DH), lambda i: (0, 0)),
            pl.BlockSpec((2, R, DEGW), lambda i: (0, i, 0)),
        ],
        out_specs=[
            pl.BlockSpec((R, DH), lambda i: (i, 0)),
            pl.BlockSpec((R, 1), lambda i: (i, 0)),
        ],
        out_shape=[
            jax.ShapeDtypeStruct((N, DH), jnp.float32),
            jax.ShapeDtypeStruct((N, 1), jnp.float32),
        ],
    )(x, W1, degout)

    acc1 = _agg_kernel(n_pad, DH)(hs1, s3, d3, z_h)

    hs2 = pl.pallas_call(
        _stage2_body,
        grid=(NG,),
        in_specs=[
            pl.BlockSpec((2, R, DH), lambda i: (0, i, 0)),
            pl.BlockSpec((R, DH), lambda i: (i, 0)),
            pl.BlockSpec((R, 1), lambda i: (i, 0)),
            pl.BlockSpec((DH, DO), lambda i: (0, 0)),
            pl.BlockSpec((1, DH), lambda i: (0, 0)),
        ],
        out_specs=pl.BlockSpec((R, DO), lambda i: (i, 0)),
        out_shape=jax.ShapeDtypeStruct((N, DO), jnp.float32),
    )(acc1, hs1, dinv, W2, b1.reshape(1, DH))

    acc2 = _agg_kernel(n_pad, DO)(hs2, s3, d3, z_o)

    out = pl.pallas_call(
        functools.partial(_stage3_body, float(N), NG),
        grid=(NG,),
        in_specs=[
            pl.BlockSpec((2, R, DO), lambda i: (0, i, 0)),
            pl.BlockSpec((R, DO), lambda i: (i, 0)),
            pl.BlockSpec((R, 1), lambda i: (i, 0)),
            pl.BlockSpec((1, DO), lambda i: (0, 0)),
            pl.BlockSpec((1, DO), lambda i: (0, 0)),
            pl.BlockSpec((1, 1), lambda i: (0, 0)),
        ],
        out_specs=pl.BlockSpec((1, 1), lambda i: (0, 0)),
        out_shape=jax.ShapeDtypeStruct((1, 1), jnp.float32),
        scratch_shapes=[pltpu.VMEM((1, DO), jnp.float32)],
    )(acc2, hs2, dinv, b2.reshape(1, DO),
      Wfc.reshape(1, DO), bfc.reshape(1, 1))

    return out


# speedup vs baseline: 1.9340x; 1.0529x over previous
import functools

import jax
import jax.numpy as jnp
from jax import lax
from jax.experimental import pallas as pl
from jax.experimental.pallas import tpu as pltpu
from jax.experimental.pallas import tpu_sc as plsc

NC = 2
NS = 16
NW = NC * NS
CH = 128
DEGW = 16

K0 = 110
K1 = 48
KH0 = K0 // 2
KH1 = K1 // 2
KHM = max(KH0, KH1)
KR = 2 * KHM


def _mesh():
    return plsc.VectorSubcoreMesh(core_axis_name="c", subcore_axis_name="s")


def _deg_kernel(n_pad):
    RPT = n_pad // NS

    @functools.partial(
        pl.kernel,
        mesh=_mesh(),
        compiler_params=pltpu.CompilerParams(use_tc_tiling_on_sc=False),
        out_type=jax.ShapeDtypeStruct((NC, n_pad, DEGW), jnp.float32),
        scratch_types=[
            pltpu.VMEM((KR, CH), jnp.int32),
            pltpu.VMEM((CH, DEGW), jnp.float32),
            pltpu.VMEM_SHARED((n_pad, DEGW), jnp.float32),
        ],
    )
    def deg_k(didx_hbm, ones_hbm, zeros_hbm, out_hbm, didx_v, ones_v, deg_sh):
        c = lax.axis_index("c")
        s = lax.axis_index("s")
        wid = s * NC + c
        kc = jnp.where(c == 0, K0, K1)
        pltpu.sync_copy(ones_hbm, ones_v)
        pltpu.sync_copy(zeros_hbm, deg_sh.at[pl.ds(s * RPT, RPT)])
        pltpu.sync_copy(didx_hbm.at[wid], didx_v)
        plsc.subcore_barrier()

        def step(k, carry):
            pltpu.sync_copy(ones_v, deg_sh.at[didx_v.at[k]], add=True)
            return carry

        lax.fori_loop(0, kc, step, 0)
        plsc.subcore_barrier()
        pltpu.sync_copy(deg_sh.at[pl.ds(s * RPT, RPT)],
                        out_hbm.at[c, pl.ds(s * RPT, RPT)])

    return deg_k


def _agg_kernel(n_pad, D):
    RPT = n_pad // NS

    @functools.partial(
        pl.kernel,
        mesh=_mesh(),
        compiler_params=pltpu.CompilerParams(use_tc_tiling_on_sc=False),
        out_type=jax.ShapeDtypeStruct((NC, n_pad, D), jnp.float32),
        scratch_types=[
            pltpu.VMEM((KHM, CH), jnp.int32),
            pltpu.VMEM((KHM, CH), jnp.int32),
            pltpu.VMEM((CH, D), jnp.float32),
            pltpu.VMEM_SHARED((n_pad, D), jnp.float32),
            pltpu.SemaphoreType.DMA,
        ],
    )
    def agg_k(tab_hbm, sidx_hbm, didx_hbm, zeros_hbm, out_hbm,
              sidx_v, didx_v, rows_v, acc_sh, gsem):
        c = lax.axis_index("c")
        s = lax.axis_index("s")
        wid = s * NC + c
        khc = jnp.where(c == 0, KH0, KH1)
        pltpu.sync_copy(zeros_hbm, acc_sh.at[pl.ds(s * RPT, RPT)])
        plsc.subcore_barrier()

        for ph in range(2):
            pltpu.sync_copy(sidx_hbm.at[wid, pl.ds(ph * khc, KHM)], sidx_v)
            pltpu.sync_copy(didx_hbm.at[wid, pl.ds(ph * khc, KHM)], didx_v)

            def step(k, carry):
                pltpu.async_copy(tab_hbm.at[sidx_v.at[k]], rows_v,
                                 gsem).wait()
                pltpu.sync_copy(rows_v, acc_sh.at[didx_v.at[k]], add=True)
                return carry

            lax.fori_loop(0, khc, step, 0)

        plsc.subcore_barrier()
        pltpu.sync_copy(acc_sh.at[pl.ds(s * RPT, RPT)],
                        out_hbm.at[c, pl.ds(s * RPT, RPT)])

    return agg_k


def _stage1_body(x_ref, w_ref, deg_ref, hs_ref, dinv_ref):
    deg = deg_ref[0, :, 0:1] + deg_ref[1, :, 0:1] + 1.0
    dinv = lax.rsqrt(deg)
    h = jnp.dot(x_ref[...], w_ref[...], preferred_element_type=jnp.float32)
    hs_ref[...] = h * dinv
    dinv_ref[...] = dinv


def _stage2_body(acc_ref, hs1_ref, dinv_ref, w_ref, b_ref, hs2_ref):
    dinv = dinv_ref[...]
    z = (acc_ref[0] + acc_ref[1]) * dinv + hs1_ref[...] * dinv + b_ref[...]
    z = jnp.maximum(z, 0.0)
    hs2_ref[...] = jnp.dot(z, w_ref[...], preferred_element_type=jnp.float32) * dinv


def _stage3_body(n_rows, n_grid,
                 acc_ref, hs2_ref, dinv_ref, b_ref, wfcT_ref,
                 bfc_ref, out_ref, acc_scr):
    i = pl.program_id(0)

    @pl.when(i == 0)
    def _():
        acc_scr[...] = jnp.zeros_like(acc_scr)

    dinv = dinv_ref[...]
    z = (acc_ref[0] + acc_ref[1]) * dinv + hs2_ref[...] * dinv + b_ref[...]
    z = jnp.maximum(z, 0.0)
    acc_scr[...] += jnp.sum(z, axis=0, keepdims=True)

    @pl.when(i == n_grid - 1)
    def _():
        g = acc_scr[...] * (1.0 / n_rows)
        out_ref[...] = (jnp.sum(g * wfcT_ref[...], axis=1, keepdims=True)
                        + bfc_ref[...])


def kernel(x, edge_index, W1, b1, W2, b2, Wfc, bfc):
    N, D_IN = x.shape
    DH = W1.shape[1]
    DO = W2.shape[1]
    E = edge_index.shape[1]

    E_cap = NS * (K0 + K1) * CH
    pad = E_cap - E
    src = jnp.concatenate([edge_index[0], jnp.zeros((pad,), jnp.int32)])
    dst = jnp.concatenate([edge_index[1], jnp.full((pad,), N, jnp.int32)])
    n0 = NS * K0 * CH

    def _layout(v, fill):
        p0 = v[:n0].reshape(NS, 1, K0 * CH)
        p1 = v[n0:].reshape(NS, 1, K1 * CH)
        if K0 < KR:
            p0 = jnp.concatenate(
                [p0, jnp.full((NS, 1, (KR - K0) * CH), fill, jnp.int32)], 2)
        if K1 < KR:
            p1 = jnp.concatenate(
                [p1, jnp.full((NS, 1, (KR - K1) * CH), fill, jnp.int32)], 2)
        return jnp.concatenate([p0, p1], 1).reshape(NW, KR, CH)

    s3 = _layout(src, 0)
    d3 = _layout(dst, N)

    RPT = -(-(N + 1) // (NS * 8)) * 8
    n_pad = RPT * NS

    ones_deg = jnp.ones((CH, DEGW), jnp.float32)
    z_deg = jnp.zeros((RPT, DEGW), jnp.float32)
    z_h = jnp.zeros((RPT, DH), jnp.float32)
    z_o = jnp.zeros((RPT, DO), jnp.float32)

    degout = _deg_kernel(n_pad)(d3, ones_deg, z_deg)

    R = 2000
    NG = N // R
    hs1, dinv = pl.pallas_call(
        _stage1_body,
        grid=(NG,),
        in_specs=[
            pl.BlockSpec((R, D_IN), lambda i: (i, 0)),
            pl.BlockSpec((D_IN, DH), lambda i: (0, 0)),
            pl.BlockSpec((2, R, DEGW), lambda i: (0, i, 0)),
        ],
        out_specs=[
            pl.BlockSpec((R, DH), lambda i: (i, 0)),
            pl.BlockSpec((R, 1), lambda i: (i, 0)),
        ],
        out_shape=[
            jax.ShapeDtypeStruct((N, DH), jnp.float32),
            jax.ShapeDtypeStruct((N, 1), jnp.float32),
        ],
    )(x, W1, degout)

    acc1 = _agg_kernel(n_pad, DH)(hs1, s3, d3, z_h)

    hs2 = pl.pallas_call(
        _stage2_body,
        grid=(NG,),
        in_specs=[
            pl.BlockSpec((2, R, DH), lambda i: (0, i, 0)),
            pl.BlockSpec((R, DH), lambda i: (i, 0)),
            pl.BlockSpec((R, 1), lambda i: (i, 0)),
            pl.BlockSpec((DH, DO), lambda i: (0, 0)),
            pl.BlockSpec((1, DH), lambda i: (0, 0)),
        ],
        out_specs=pl.BlockSpec((R, DO), lambda i: (i, 0)),
        out_shape=jax.ShapeDtypeStruct((N, DO), jnp.float32),
    )(acc1, hs1, dinv, W2, b1.reshape(1, DH))

    acc2 = _agg_kernel(n_pad, DO)(hs2, s3, d3, z_o)

    out = pl.pallas_call(
        functools.partial(_stage3_body, float(N), NG),
        grid=(NG,),
        in_specs=[
            pl.BlockSpec((2, R, DO), lambda i: (0, i, 0)),
            pl.BlockSpec((R, DO), lambda i: (i, 0)),
            pl.BlockSpec((R, 1), lambda i: (i, 0)),
            pl.BlockSpec((1, DO), lambda i: (0, 0)),
            pl.BlockSpec((1, DO), lambda i: (0, 0)),
            pl.BlockSpec((1, 1), lambda i: (0, 0)),
        ],
        out_specs=pl.BlockSpec((1, 1), lambda i: (0, 0)),
        out_shape=jax.ShapeDtypeStruct((1, 1), jnp.float32),
        scratch_shapes=[pltpu.VMEM((1, DO), jnp.float32)],
    )(acc2, hs2, dinv, b2.reshape(1, DO),
      Wfc.reshape(1, DO), bfc.reshape(1, 1))

    return out
